# baseline (speedup 1.0000x reference)
import functools

import jax
import jax.numpy as jnp
from jax import lax
from jax.experimental import pallas as pl
from jax.experimental.pallas import tpu as pltpu
from jax.experimental.pallas import tpu_sc as plsc

NC = 2
NS = 16
NW = NC * NS

CHUNK_IDX = 1024


def _make_kernel(n_idx, emb_dim, n_hash):
    assert n_idx % (NW * CHUNK_IDX) == 0
    idx_per_w = n_idx // NW
    chunks = idx_per_w // CHUNK_IDX
    assert chunks >= 4 and chunks % 2 == 0
    out_per_chunk = CHUNK_IDX // n_hash
    packed_per_chunk = out_per_chunk * emb_dim // 128
    n_out = n_idx // n_hash
    half = emb_dim // 2

    mesh = plsc.VectorSubcoreMesh(
        core_axis_name="c", subcore_axis_name="s", num_cores=NC, num_subcores=NS
    )

    @functools.partial(
        pl.kernel,
        out_type=jax.ShapeDtypeStruct((n_out * emb_dim // 128, 128), jnp.float32),
        mesh=mesh,
        scratch_types=[
            pltpu.VMEM((2, 8, CHUNK_IDX // 8), jnp.int32),
            pltpu.VMEM((2, CHUNK_IDX, emb_dim), jnp.float32),
            pltpu.VMEM((2, packed_per_chunk, 128), jnp.float32),
            pltpu.SemaphoreType.DMA,
            pltpu.SemaphoreType.DMA,
            pltpu.SemaphoreType.DMA,
            pltpu.SemaphoreType.DMA,
        ],
        compiler_params=pltpu.CompilerParams(
            use_tc_tiling_on_sc=False, needs_layout_passes=False
        ),
    )
    def k(x_hbm, e_hbm, y_hbm, idx_v, rows_v, out_v, isem, gsem, osem0, osem1):
        wid = lax.axis_index("s") * NC + lax.axis_index("c")
        idx_row0 = wid * (idx_per_w // 128)
        blk0 = wid * (idx_per_w // 512)
        osems = (osem0, osem1)

        def idx_load(c, s):
            return pltpu.async_copy(
                x_hbm.at[pl.ds(idx_row0 + c * 8, 8)], idx_v.at[s], isem
            )

        def wait_idx_load(s):
            pltpu.make_async_copy(
                x_hbm.at[pl.ds(idx_row0, 8)], idx_v.at[s], isem
            ).wait()

        def gathers(s):
            for j in range(8):
                pltpu.async_copy(
                    e_hbm.at[idx_v.at[s, j]],
                    rows_v.at[s, pl.ds(j * 128, 128)],
                    gsem,
                )

        def wait_gathers(s):
            for j in range(8):
                pltpu.make_async_copy(
                    e_hbm.at[idx_v.at[s, j]],
                    rows_v.at[s, pl.ds(j * 128, 128)],
                    gsem,
                ).wait()

        def store(c, s):
            for k in range(2):
                bk = blk0 + 2 * c + k
                ll = bk // 128
                bb = bk % 128
                for r in range(4):
                    base = ((ll * 4 + r) * 128 + bb) * 8
                    pltpu.async_copy(
                        out_v.at[s, pl.ds(k * 32 + r * 8, 8)],
                        y_hbm.at[pl.ds(base, 8)],
                        osems[s],
                    )

        def wait_store(s):
            for _ in range(8):
                pltpu.make_async_copy(
                    out_v.at[s, pl.ds(0, 8)],
                    y_hbm.at[pl.ds(0, 8)],
                    osems[s],
                ).wait()

        def reduce(s):
            rv = rows_v.at[s]
            ov = out_v.at[s]

            dlo = lax.iota(jnp.int32, half)
            dhi = dlo + half

            @pl.loop(0, out_per_chunk, unroll=8)
            def _red(t):
                k = t // 128
                b = t % 128
                r = k * 512 + b
                lo = rv[r, pl.ds(0, half)]
                hi = rv[r, pl.ds(half, half)]
                for h in range(1, n_hash):
                    lo = lo + rv[r + h * 128, pl.ds(0, half)]
                    hi = hi + rv[r + h * 128, pl.ds(half, half)]
                bcol = jnp.full((half,), b, dtype=jnp.int32)
                plsc.store_scatter(ov, [dlo + k * 32, bcol], lo)
                plsc.store_scatter(ov, [dhi + k * 32, bcol], hi)

        def step(c, s, issue_next, load_next2, drain_store):
            wait_gathers(s)
            if issue_next:
                wait_idx_load(1 - s)
                gathers(1 - s)
            if load_next2:
                idx_load(c + 2, s)
            if drain_store:
                wait_store(s)
            reduce(s)
            store(c, s)

        idx_load(0, 0)
        wait_idx_load(0)
        gathers(0)
        idx_load(1, 1)
        step(0, 0, True, True, False)
        step(1, 1, True, True, False)

        @pl.loop(1, (chunks - 4) // 2 + 1)
        def _main(i):
            step(2 * i, 0, True, True, True)
            step(2 * i + 1, 1, True, True, True)

        step(chunks - 2, 0, True, False, True)
        step(chunks - 1, 1, False, False, True)
        wait_store(0)
        wait_store(1)

    return k


def kernel(x, E):
    b, l, h = x.shape
    n_tok, emb_dim = E.shape
    n_idx = b * l * h
    x2d = (
        x.reshape(b // 128, 128, l, h)
        .transpose(2, 0, 3, 1)
        .reshape(n_idx // 128, 128)
        .astype(jnp.int32)
    )
    y = _make_kernel(n_idx, emb_dim, h)(x2d, E)
    y5 = y.reshape(l, emb_dim // 8, b // 128, 8, 128)
    return y5.transpose(2, 4, 0, 1, 3).reshape(b, l, emb_dim)


# speedup vs baseline: 1.4455x; 1.4455x over previous
import functools

import jax
import jax.numpy as jnp
from jax import lax
from jax.experimental import pallas as pl
from jax.experimental.pallas import tpu as pltpu
from jax.experimental.pallas import tpu_sc as plsc

NC = 2
NS = 16
NW = NC * NS

CHUNK_IDX = 1024


def _make_kernel(n_idx, emb_dim, n_hash):
    assert n_idx % (NW * CHUNK_IDX) == 0
    idx_per_w = n_idx // NW
    chunks = idx_per_w // CHUNK_IDX
    assert chunks >= 4 and chunks % 2 == 0
    out_per_chunk = CHUNK_IDX // n_hash
    packed_per_chunk = out_per_chunk * emb_dim // 128
    n_out = n_idx // n_hash
    half = emb_dim // 2

    mesh = plsc.VectorSubcoreMesh(
        core_axis_name="c", subcore_axis_name="s", num_cores=NC, num_subcores=NS
    )

    @functools.partial(
        pl.kernel,
        out_type=jax.ShapeDtypeStruct((n_out * emb_dim // 128, 128), jnp.float32),
        mesh=mesh,
        scratch_types=[
            pltpu.VMEM((2, 8, CHUNK_IDX // 8), jnp.int32),
            pltpu.VMEM((2, CHUNK_IDX, emb_dim), jnp.float32),
            pltpu.VMEM((2, packed_per_chunk, 129), jnp.float32),
            pltpu.SemaphoreType.DMA,
            pltpu.SemaphoreType.DMA,
            pltpu.SemaphoreType.DMA,
            pltpu.SemaphoreType.DMA,
        ],
        compiler_params=pltpu.CompilerParams(
            use_tc_tiling_on_sc=False, needs_layout_passes=False
        ),
    )
    def k(x_hbm, e_hbm, y_hbm, idx_v, rows_v, out_v, isem, gsem, osem0, osem1):
        wid = lax.axis_index("s") * NC + lax.axis_index("c")
        idx_row0 = wid * (idx_per_w // 128)
        blk0 = wid * (idx_per_w // 512)
        osems = (osem0, osem1)

        def idx_load(c, s):
            return pltpu.async_copy(
                x_hbm.at[pl.ds(idx_row0 + c * 8, 8)], idx_v.at[s], isem
            )

        def wait_idx_load(s):
            pltpu.make_async_copy(
                x_hbm.at[pl.ds(idx_row0, 8)], idx_v.at[s], isem
            ).wait()

        def gathers(s):
            for j in range(8):
                pltpu.async_copy(
                    e_hbm.at[idx_v.at[s, j]],
                    rows_v.at[s, pl.ds(j * 128, 128)],
                    gsem,
                )

        def wait_gathers(s):
            for j in range(8):
                pltpu.make_async_copy(
                    e_hbm.at[idx_v.at[s, j]],
                    rows_v.at[s, pl.ds(j * 128, 128)],
                    gsem,
                ).wait()

        def store(c, s):
            for k in range(2):
                bk = blk0 + 2 * c + k
                ll = bk // 128
                bb = bk % 128
                for r in range(4):
                    base = ((ll * 4 + r) * 128 + bb) * 8
                    pltpu.async_copy(
                        out_v.at[s, pl.ds(k * 32 + r * 8, 8), pl.ds(0, 128)],
                        y_hbm.at[pl.ds(base, 8)],
                        osems[s],
                    )

        def wait_store(s):
            for _ in range(8):
                pltpu.make_async_copy(
                    out_v.at[s, pl.ds(0, 8), pl.ds(0, 128)],
                    y_hbm.at[pl.ds(0, 8)],
                    osems[s],
                ).wait()

        def reduce(s):
            rv = rows_v.at[s]
            ov = out_v.at[s]

            dlo = lax.iota(jnp.int32, half)
            dhi = dlo + half

            @pl.loop(0, out_per_chunk, unroll=8)
            def _red(t):
                k = t // 128
                b = t % 128
                r = k * 512 + b
                lo = rv[r, pl.ds(0, half)]
                hi = rv[r, pl.ds(half, half)]
                for h in range(1, n_hash):
                    lo = lo + rv[r + h * 128, pl.ds(0, half)]
                    hi = hi + rv[r + h * 128, pl.ds(half, half)]
                bcol = jnp.full((half,), b, dtype=jnp.int32)
                plsc.store_scatter(ov, [dlo + k * 32, bcol], lo)
                plsc.store_scatter(ov, [dhi + k * 32, bcol], hi)

        def step(c, s, issue_next, load_next2, drain_store):
            wait_gathers(s)
            if issue_next:
                wait_idx_load(1 - s)
                gathers(1 - s)
            if load_next2:
                idx_load(c + 2, s)
            if drain_store:
                wait_store(s)
            reduce(s)
            store(c, s)

        idx_load(0, 0)
        wait_idx_load(0)
        gathers(0)
        idx_load(1, 1)
        step(0, 0, True, True, False)
        step(1, 1, True, True, False)

        @pl.loop(1, (chunks - 4) // 2 + 1)
        def _main(i):
            step(2 * i, 0, True, True, True)
            step(2 * i + 1, 1, True, True, True)

        step(chunks - 2, 0, True, False, True)
        step(chunks - 1, 1, False, False, True)
        wait_store(0)
        wait_store(1)

    return k


def kernel(x, E):
    b, l, h = x.shape
    n_tok, emb_dim = E.shape
    n_idx = b * l * h
    x2d = (
        x.reshape(b // 128, 128, l, h)
        .transpose(2, 0, 3, 1)
        .reshape(n_idx // 128, 128)
        .astype(jnp.int32)
    )
    y = _make_kernel(n_idx, emb_dim, h)(x2d, E)
    y5 = y.reshape(l, emb_dim // 8, b // 128, 8, 128)
    return y5.transpose(2, 4, 0, 1, 3).reshape(b, l, emb_dim)
